# Initial kernel scaffold; baseline (speedup 1.0000x reference)
import jax
import jax.numpy as jnp
from jax.experimental import pallas as pl


def kernel(x, edge_index, edge_weight, W1, Wmu, Wlogvar):
    raise NotImplementedError("write your pallas kernel here")


# speedup vs baseline: 8.5212x; 8.5212x over previous
import functools

import jax
import jax.numpy as jnp
from jax import lax
from jax.experimental import pallas as pl
from jax.experimental.pallas import tpu as pltpu
from jax.experimental.pallas import tpu_sc as plsc

NC = 2
NS = 16
NW = NC * NS
LANES = 16

CHUNK = 512
IDX_ROWS = CHUNK // 128


def _spmm_body(h_hbm, src_hbm, dst_hbm, w_hbm, zeros_hbm, out_hbm,
               idx_v, dst_v, w_v, rows_v, acc_sh, sem, *, n, f, e):
  cid = lax.axis_index("c")
  sid = lax.axis_index("s")
  wid = cid * NS + sid
  epw = e // NW
  nchunks = epw // CHUNK
  zrows = n // NS

  pltpu.sync_copy(zeros_hbm.at[pl.ds(sid * zrows, zrows)],
                  acc_sh.at[pl.ds(sid * zrows, zrows)])
  plsc.subcore_barrier()

  @pl.loop(0, nchunks)
  def _chunk(k):
    crow0 = wid * (epw // 128) + k * IDX_ROWS
    pltpu.sync_copy(src_hbm.at[pl.ds(crow0, IDX_ROWS)], idx_v)
    pltpu.sync_copy(dst_hbm.at[pl.ds(crow0, IDX_ROWS)], dst_v)
    pltpu.sync_copy(w_hbm.at[pl.ds(crow0, IDX_ROWS)], w_v)
    descs = [
        pltpu.async_copy(h_hbm.at[idx_v.at[j]],
                         rows_v.at[pl.ds(j * 128, 128)], sem)
        for j in range(IDX_ROWS)
    ]
    for d in descs:
      d.wait()

    @pl.loop(0, CHUNK // LANES)
    def _blk(b):
      wrow = lax.shift_right_logical(b, 3)
      wcol = lax.bitwise_and(b, 7) * LANES
      wvec = w_v[wrow, pl.ds(wcol, LANES)]
      for i in range(LANES):
        ei = b * LANES + i
        wv = jnp.full((LANES,), wvec[i], jnp.float32)
        for half in range(f // LANES):
          sl = pl.ds(half * LANES, LANES)
          rows_v[ei, sl] = rows_v[ei, sl] * wv

    for j in range(IDX_ROWS):
      pltpu.sync_copy(rows_v.at[pl.ds(j * 128, 128)],
                      acc_sh.at[dst_v.at[j]], add=True)

  plsc.subcore_barrier()
  pltpu.sync_copy(acc_sh.at[pl.ds(sid * zrows, zrows)],
                  rows_v.at[pl.ds(0, zrows)])
  pltpu.sync_copy(rows_v.at[pl.ds(0, zrows)],
                  out_hbm.at[cid, pl.ds(sid * zrows, zrows)])


def _spmm_partials(h, src2d, dst2d, w2d, zeros):
  n, f = h.shape
  e = src2d.shape[0] * src2d.shape[1]
  mesh = plsc.VectorSubcoreMesh(core_axis_name="c", subcore_axis_name="s")
  body = functools.partial(_spmm_body, n=n, f=f, e=e)
  run = pl.kernel(
      body,
      out_type=jax.ShapeDtypeStruct((NC, n, f), jnp.float32),
      mesh=mesh,
      scratch_types=[
          pltpu.VMEM((IDX_ROWS, 128), jnp.int32),
          pltpu.VMEM((IDX_ROWS, 128), jnp.int32),
          pltpu.VMEM((IDX_ROWS, 128), jnp.float32),
          pltpu.VMEM((CHUNK, f), jnp.float32),
          pltpu.VMEM_SHARED((n, f), jnp.float32),
          pltpu.SemaphoreType.DMA,
      ],
      compiler_params=pltpu.CompilerParams(use_tc_tiling_on_sc=False),
  )
  return run(h, src2d, dst2d, w2d, zeros)


def _mm_kernel(x_ref, w_ref, o_ref):
  o_ref[...] = jnp.dot(x_ref[...], w_ref[...],
                       preferred_element_type=jnp.float32)


def _matmul(x, w, blk):
  n, d = x.shape
  h = w.shape[1]
  return pl.pallas_call(
      _mm_kernel,
      grid=(n // blk,),
      in_specs=[
          pl.BlockSpec((blk, d), lambda i: (i, 0)),
          pl.BlockSpec((d, h), lambda i: (0, 0)),
      ],
      out_specs=pl.BlockSpec((blk, h), lambda i: (i, 0)),
      out_shape=jax.ShapeDtypeStruct((n, h), jnp.float32),
  )(x, w)


def _relu_sum_kernel(p_ref, o_ref):
  o_ref[...] = jnp.maximum(p_ref[0] + p_ref[1], 0.0)


def _relu_sum(p, blk=1024):
  _, n, f = p.shape
  return pl.pallas_call(
      _relu_sum_kernel,
      grid=(n // blk,),
      in_specs=[pl.BlockSpec((2, blk, f), lambda i: (0, i, 0))],
      out_specs=pl.BlockSpec((blk, f), lambda i: (i, 0)),
      out_shape=jax.ShapeDtypeStruct((n, f), jnp.float32),
  )(p)


def _heads_kernel(q_ref, wmu_ref, wlv_ref, mu_ref, lv_ref):
  s = q_ref[0] + q_ref[1]
  mu_ref[...] = jnp.dot(s, wmu_ref[...], preferred_element_type=jnp.float32)
  lv_ref[...] = jnp.dot(s, wlv_ref[...], preferred_element_type=jnp.float32)


def _heads(q, wmu, wlv, blk=1024):
  _, n, f = q.shape
  h2 = wmu.shape[1]
  return pl.pallas_call(
      _heads_kernel,
      grid=(n // blk,),
      in_specs=[
          pl.BlockSpec((2, blk, f), lambda i: (0, i, 0)),
          pl.BlockSpec((f, h2), lambda i: (0, 0)),
          pl.BlockSpec((f, h2), lambda i: (0, 0)),
      ],
      out_specs=[
          pl.BlockSpec((blk, h2), lambda i: (i, 0)),
          pl.BlockSpec((blk, h2), lambda i: (i, 0)),
      ],
      out_shape=[
          jax.ShapeDtypeStruct((n, h2), jnp.float32),
          jax.ShapeDtypeStruct((n, h2), jnp.float32),
      ],
  )(q, wmu, wlv)


def _decode_kernel(zi_ref, zj_ref, o_ref):
  o_ref[...] = lax.dot_general(
      zi_ref[...], zj_ref[...], (((1,), (1,)), ((), ())),
      preferred_element_type=jnp.float32)


def _decode(z, blk_i=512, blk_j=2048):
  n, h2 = z.shape
  return pl.pallas_call(
      _decode_kernel,
      grid=(n // blk_i, n // blk_j),
      in_specs=[
          pl.BlockSpec((blk_i, h2), lambda i, j: (i, 0)),
          pl.BlockSpec((blk_j, h2), lambda i, j: (j, 0)),
      ],
      out_specs=pl.BlockSpec((blk_i, blk_j), lambda i, j: (i, j)),
      out_shape=jax.ShapeDtypeStruct((n, n), jnp.float32),
  )(z, z)


def kernel(x, edge_index, edge_weight, W1, Wmu, Wlogvar):
  n = x.shape[0]
  src2d = edge_index[0].reshape(-1, 128)
  dst2d = edge_index[1].reshape(-1, 128)
  w2d = edge_weight.reshape(-1, 128)
  zeros = jnp.zeros((n, W1.shape[1]), jnp.float32)

  h0 = _matmul(x, W1, blk=512)
  p = _spmm_partials(h0, src2d, dst2d, w2d, zeros)
  h1 = _relu_sum(p)
  q = _spmm_partials(h1, src2d, dst2d, w2d, zeros)
  mu, logvar = _heads(q, Wmu, Wlogvar)
  adj_hat = _decode(mu)
  return (adj_hat, mu, logvar)


# speedup vs baseline: 9.5791x; 1.1241x over previous
import functools

import jax
import jax.numpy as jnp
from jax import lax
from jax.experimental import pallas as pl
from jax.experimental.pallas import tpu as pltpu
from jax.experimental.pallas import tpu_sc as plsc

NC = 2
NS = 16
NW = NC * NS
LANES = 16

CHUNK = 512
IDX_ROWS = CHUNK // 128


def _spmm_body(h_hbm, src_hbm, dst_hbm, w_hbm, zeros_hbm, out_hbm,
               idx_v, dst_v, w_v, rows_v, acc_sh, sem, *, n, f, e):
  cid = lax.axis_index("c")
  sid = lax.axis_index("s")
  wid = cid * NS + sid
  epw = e // NW
  nchunks = epw // CHUNK
  zrows = n // NS

  pltpu.sync_copy(zeros_hbm.at[pl.ds(sid * zrows, zrows)],
                  acc_sh.at[pl.ds(sid * zrows, zrows)])
  plsc.subcore_barrier()

  @pl.loop(0, nchunks)
  def _chunk(k):
    crow0 = wid * (epw // 128) + k * IDX_ROWS
    pltpu.sync_copy(src_hbm.at[pl.ds(crow0, IDX_ROWS)], idx_v)
    pltpu.sync_copy(dst_hbm.at[pl.ds(crow0, IDX_ROWS)], dst_v)
    pltpu.sync_copy(w_hbm.at[pl.ds(crow0, IDX_ROWS)], w_v)
    descs = [
        pltpu.async_copy(h_hbm.at[idx_v.at[j]],
                         rows_v.at[pl.ds(j * 128, 128)], sem)
        for j in range(IDX_ROWS)
    ]
    for d in descs:
      d.wait()

    @pl.loop(0, CHUNK // LANES)
    def _blk(b):
      wrow = lax.shift_right_logical(b, 3)
      wcol = lax.bitwise_and(b, 7) * LANES
      wvec = w_v[wrow, pl.ds(wcol, LANES)]
      for i in range(LANES):
        ei = b * LANES + i
        wv = jnp.full((LANES,), wvec[i], jnp.float32)
        for half in range(f // LANES):
          sl = pl.ds(half * LANES, LANES)
          rows_v[ei, sl] = rows_v[ei, sl] * wv

    for j in range(IDX_ROWS):
      pltpu.sync_copy(rows_v.at[pl.ds(j * 128, 128)],
                      acc_sh.at[dst_v.at[j]], add=True)

  plsc.subcore_barrier()
  pltpu.sync_copy(acc_sh.at[pl.ds(sid * zrows, zrows)],
                  rows_v.at[pl.ds(0, zrows)])
  pltpu.sync_copy(rows_v.at[pl.ds(0, zrows)],
                  out_hbm.at[cid, pl.ds(sid * zrows, zrows)])


def _spmm_partials(h, src2d, dst2d, w2d, zeros):
  n, f = h.shape
  e = src2d.shape[0] * src2d.shape[1]
  mesh = plsc.VectorSubcoreMesh(core_axis_name="c", subcore_axis_name="s")
  body = functools.partial(_spmm_body, n=n, f=f, e=e)
  run = pl.kernel(
      body,
      out_type=jax.ShapeDtypeStruct((NC, n, f), jnp.float32),
      mesh=mesh,
      scratch_types=[
          pltpu.VMEM((IDX_ROWS, 128), jnp.int32),
          pltpu.VMEM((IDX_ROWS, 128), jnp.int32),
          pltpu.VMEM((IDX_ROWS, 128), jnp.float32),
          pltpu.VMEM((CHUNK, f), jnp.float32),
          pltpu.VMEM_SHARED((n, f), jnp.float32),
          pltpu.SemaphoreType.DMA,
      ],
      compiler_params=pltpu.CompilerParams(use_tc_tiling_on_sc=False),
  )
  return run(h, src2d, dst2d, w2d, zeros)


def _mm_kernel(x_ref, w_ref, o_ref):
  o_ref[...] = jnp.dot(x_ref[...], w_ref[...],
                       preferred_element_type=jnp.float32)


def _matmul(x, w, blk):
  n, d = x.shape
  h = w.shape[1]
  return pl.pallas_call(
      _mm_kernel,
      grid=(n // blk,),
      in_specs=[
          pl.BlockSpec((blk, d), lambda i: (i, 0)),
          pl.BlockSpec((d, h), lambda i: (0, 0)),
      ],
      out_specs=pl.BlockSpec((blk, h), lambda i: (i, 0)),
      out_shape=jax.ShapeDtypeStruct((n, h), jnp.float32),
  )(x, w)


def _relu_sum_kernel(p_ref, o_ref):
  o_ref[...] = jnp.maximum(p_ref[0] + p_ref[1], 0.0)


def _relu_sum(p, blk=8192):
  _, n, f = p.shape
  return pl.pallas_call(
      _relu_sum_kernel,
      grid=(n // blk,),
      in_specs=[pl.BlockSpec((2, blk, f), lambda i: (0, i, 0))],
      out_specs=pl.BlockSpec((blk, f), lambda i: (i, 0)),
      out_shape=jax.ShapeDtypeStruct((n, f), jnp.float32),
  )(p)


def _heads_kernel(q_ref, wmu_ref, wlv_ref, mu_ref, lv_ref):
  s = q_ref[0] + q_ref[1]
  mu_ref[...] = jnp.dot(s, wmu_ref[...], preferred_element_type=jnp.float32)
  lv_ref[...] = jnp.dot(s, wlv_ref[...], preferred_element_type=jnp.float32)


def _heads(q, wmu, wlv, blk=8192):
  _, n, f = q.shape
  h2 = wmu.shape[1]
  return pl.pallas_call(
      _heads_kernel,
      grid=(n // blk,),
      in_specs=[
          pl.BlockSpec((2, blk, f), lambda i: (0, i, 0)),
          pl.BlockSpec((f, h2), lambda i: (0, 0)),
          pl.BlockSpec((f, h2), lambda i: (0, 0)),
      ],
      out_specs=[
          pl.BlockSpec((blk, h2), lambda i: (i, 0)),
          pl.BlockSpec((blk, h2), lambda i: (i, 0)),
      ],
      out_shape=[
          jax.ShapeDtypeStruct((n, h2), jnp.float32),
          jax.ShapeDtypeStruct((n, h2), jnp.float32),
      ],
  )(q, wmu, wlv)


def _decode_kernel(zi_ref, zj_ref, o_ref):
  o_ref[...] = lax.dot_general(
      zi_ref[...], zj_ref[...], (((1,), (1,)), ((), ())),
      preferred_element_type=jnp.float32)


def _decode(z, blk_i=1024, blk_j=4096):
  n, h2 = z.shape
  return pl.pallas_call(
      _decode_kernel,
      grid=(n // blk_i, n // blk_j),
      in_specs=[
          pl.BlockSpec((blk_i, h2), lambda i, j: (i, 0)),
          pl.BlockSpec((blk_j, h2), lambda i, j: (j, 0)),
      ],
      out_specs=pl.BlockSpec((blk_i, blk_j), lambda i, j: (i, j)),
      out_shape=jax.ShapeDtypeStruct((n, n), jnp.float32),
  )(z, z)


def kernel(x, edge_index, edge_weight, W1, Wmu, Wlogvar):
  n = x.shape[0]
  src2d = edge_index[0].reshape(-1, 128)
  dst2d = edge_index[1].reshape(-1, 128)
  w2d = edge_weight.reshape(-1, 128)
  zeros = jnp.zeros((n, W1.shape[1]), jnp.float32)

  h0 = _matmul(x, W1, blk=2048)
  p = _spmm_partials(h0, src2d, dst2d, w2d, zeros)
  h1 = _relu_sum(p)
  q = _spmm_partials(h1, src2d, dst2d, w2d, zeros)
  mu, logvar = _heads(q, Wmu, Wlogvar)
  adj_hat = _decode(mu)
  return (adj_hat, mu, logvar)


# speedup vs baseline: 11.6500x; 1.2162x over previous
import functools

import jax
import jax.numpy as jnp
from jax import lax
from jax.experimental import pallas as pl
from jax.experimental.pallas import tpu as pltpu
from jax.experimental.pallas import tpu_sc as plsc

NC = 2
NS = 16
NW = NC * NS
LANES = 16

CHUNK = 512
IDX_ROWS = CHUNK // 128


def _spmm_body(h_hbm, src_hbm, dst_hbm, w_hbm, zeros_hbm, out_hbm,
               idx_v, dst_v, w_v, rows_v, acc_sh, sem_z, sem_i, sem_g, sem_s,
               *, n, f, e):
  cid = lax.axis_index("c")
  sid = lax.axis_index("s")
  wid = cid * NS + sid
  epw = e // NW
  nchunks = epw // CHUNK
  rows_per_w = epw // 128
  zrows = n // NS

  zd = pltpu.async_copy(zeros_hbm.at[pl.ds(sid * zrows, zrows)],
                        acc_sh.at[pl.ds(sid * zrows, zrows)], sem_z)
  crow0 = wid * rows_per_w
  ids = [
      pltpu.async_copy(src_hbm.at[pl.ds(crow0, rows_per_w)], idx_v, sem_i),
      pltpu.async_copy(dst_hbm.at[pl.ds(crow0, rows_per_w)], dst_v, sem_i),
      pltpu.async_copy(w_hbm.at[pl.ds(crow0, rows_per_w)], w_v, sem_i),
  ]
  for d in ids:
    d.wait()

  def fire_gather(k):
    buf = k % 2
    return [
        pltpu.async_copy(h_hbm.at[idx_v.at[k * IDX_ROWS + j]],
                         rows_v.at[buf, pl.ds(j * 128, 128)], sem_g)
        for j in range(IDX_ROWS)
    ]

  def fire_scatter(k):
    buf = k % 2
    return [
        pltpu.async_copy(rows_v.at[buf, pl.ds(j * 128, 128)],
                         acc_sh.at[dst_v.at[k * IDX_ROWS + j]], sem_s,
                         add=True)
        for j in range(IDX_ROWS)
    ]

  gd = fire_gather(0)
  zd.wait()
  plsc.subcore_barrier()

  sd_prev = None
  for k in range(nchunks):
    buf = k % 2
    for d in gd:
      d.wait()
    if sd_prev is not None:
      for d in sd_prev:
        d.wait()
      sd_prev = None
    if k + 1 < nchunks:
      gd = fire_gather(k + 1)

    @pl.loop(0, CHUNK // LANES)
    def _blk(b):
      g = k * (CHUNK // LANES) + b
      wrow = lax.shift_right_logical(g, 3)
      wcol = lax.bitwise_and(g, 7) * LANES
      wvec = w_v[wrow, pl.ds(wcol, LANES)]
      for i in range(LANES):
        ei = b * LANES + i
        wv = jnp.full((LANES,), wvec[i], jnp.float32)
        for half in range(f // LANES):
          sl = pl.ds(half * LANES, LANES)
          rows_v[buf, ei, sl] = rows_v[buf, ei, sl] * wv

    sd = fire_scatter(k)
    if k == nchunks - 1:
      for d in sd:
        d.wait()
    else:
      sd_prev = sd

  plsc.subcore_barrier()
  pltpu.sync_copy(acc_sh.at[pl.ds(sid * zrows, zrows)],
                  rows_v.at[0, pl.ds(0, zrows)])
  pltpu.sync_copy(rows_v.at[0, pl.ds(0, zrows)],
                  out_hbm.at[cid, pl.ds(sid * zrows, zrows)])


def _spmm_partials(h, src2d, dst2d, w2d, zeros):
  n, f = h.shape
  e = src2d.shape[0] * src2d.shape[1]
  mesh = plsc.VectorSubcoreMesh(core_axis_name="c", subcore_axis_name="s")
  body = functools.partial(_spmm_body, n=n, f=f, e=e)
  run = pl.kernel(
      body,
      out_type=jax.ShapeDtypeStruct((NC, n, f), jnp.float32),
      mesh=mesh,
      scratch_types=[
          pltpu.VMEM((e // NW // 128, 128), jnp.int32),
          pltpu.VMEM((e // NW // 128, 128), jnp.int32),
          pltpu.VMEM((e // NW // 128, 128), jnp.float32),
          pltpu.VMEM((2, CHUNK, f), jnp.float32),
          pltpu.VMEM_SHARED((n, f), jnp.float32),
          pltpu.SemaphoreType.DMA,
          pltpu.SemaphoreType.DMA,
          pltpu.SemaphoreType.DMA,
          pltpu.SemaphoreType.DMA,
      ],
      compiler_params=pltpu.CompilerParams(use_tc_tiling_on_sc=False),
  )
  return run(h, src2d, dst2d, w2d, zeros)


def _mm_kernel(x_ref, w_ref, o_ref):
  o_ref[...] = jnp.dot(x_ref[...], w_ref[...],
                       preferred_element_type=jnp.float32)


def _matmul(x, w, blk):
  n, d = x.shape
  h = w.shape[1]
  return pl.pallas_call(
      _mm_kernel,
      grid=(n // blk,),
      in_specs=[
          pl.BlockSpec((blk, d), lambda i: (i, 0)),
          pl.BlockSpec((d, h), lambda i: (0, 0)),
      ],
      out_specs=pl.BlockSpec((blk, h), lambda i: (i, 0)),
      out_shape=jax.ShapeDtypeStruct((n, h), jnp.float32),
  )(x, w)


def _relu_sum_kernel(p_ref, o_ref):
  o_ref[...] = jnp.maximum(p_ref[0] + p_ref[1], 0.0)


def _relu_sum(p, blk=8192):
  _, n, f = p.shape
  return pl.pallas_call(
      _relu_sum_kernel,
      grid=(n // blk,),
      in_specs=[pl.BlockSpec((2, blk, f), lambda i: (0, i, 0))],
      out_specs=pl.BlockSpec((blk, f), lambda i: (i, 0)),
      out_shape=jax.ShapeDtypeStruct((n, f), jnp.float32),
  )(p)


def _heads_kernel(q_ref, wmu_ref, wlv_ref, mu_ref, lv_ref):
  s = q_ref[0] + q_ref[1]
  mu_ref[...] = jnp.dot(s, wmu_ref[...], preferred_element_type=jnp.float32)
  lv_ref[...] = jnp.dot(s, wlv_ref[...], preferred_element_type=jnp.float32)


def _heads(q, wmu, wlv, blk=8192):
  _, n, f = q.shape
  h2 = wmu.shape[1]
  return pl.pallas_call(
      _heads_kernel,
      grid=(n // blk,),
      in_specs=[
          pl.BlockSpec((2, blk, f), lambda i: (0, i, 0)),
          pl.BlockSpec((f, h2), lambda i: (0, 0)),
          pl.BlockSpec((f, h2), lambda i: (0, 0)),
      ],
      out_specs=[
          pl.BlockSpec((blk, h2), lambda i: (i, 0)),
          pl.BlockSpec((blk, h2), lambda i: (i, 0)),
      ],
      out_shape=[
          jax.ShapeDtypeStruct((n, h2), jnp.float32),
          jax.ShapeDtypeStruct((n, h2), jnp.float32),
      ],
  )(q, wmu, wlv)


def _decode_kernel(zi_ref, zj_ref, o_ref):
  o_ref[...] = lax.dot_general(
      zi_ref[...], zj_ref[...], (((1,), (1,)), ((), ())),
      preferred_element_type=jnp.float32)


def _decode(z, blk_i=1024, blk_j=4096):
  n, h2 = z.shape
  return pl.pallas_call(
      _decode_kernel,
      grid=(n // blk_i, n // blk_j),
      in_specs=[
          pl.BlockSpec((blk_i, h2), lambda i, j: (i, 0)),
          pl.BlockSpec((blk_j, h2), lambda i, j: (j, 0)),
      ],
      out_specs=pl.BlockSpec((blk_i, blk_j), lambda i, j: (i, j)),
      out_shape=jax.ShapeDtypeStruct((n, n), jnp.float32),
  )(z, z)


def kernel(x, edge_index, edge_weight, W1, Wmu, Wlogvar):
  n = x.shape[0]
  src2d = edge_index[0].reshape(-1, 128)
  dst2d = edge_index[1].reshape(-1, 128)
  w2d = edge_weight.reshape(-1, 128)
  zeros = jnp.zeros((n, W1.shape[1]), jnp.float32)

  h0 = _matmul(x, W1, blk=2048)
  p = _spmm_partials(h0, src2d, dst2d, w2d, zeros)
  h1 = _relu_sum(p)
  q = _spmm_partials(h1, src2d, dst2d, w2d, zeros)
  mu, logvar = _heads(q, Wmu, Wlogvar)
  adj_hat = _decode(mu)
  return (adj_hat, mu, logvar)
